# baseline (speedup 1.0000x reference)
import functools

import jax
import jax.numpy as jnp
from jax import lax
from jax.experimental import pallas as pl
from jax.experimental.pallas import tpu as pltpu
from jax.experimental.pallas import tpu_sc as plsc

POS_OFFSET = 2
LANES = 16


@functools.lru_cache(maxsize=None)
def _make_kernel(B, S, V, H, NC, NS):
    NW = NC * NS
    total_rows = B * S
    rows_per_w = total_rows // NW
    R = 16
    n_chunks = rows_per_w // R
    vecs_per_row = H // LANES

    mesh = plsc.VectorSubcoreMesh(core_axis_name="c", subcore_axis_name="s")

    @functools.partial(
        pl.kernel,
        mesh=mesh,
        out_type=jax.ShapeDtypeStruct((total_rows, H), jnp.float32),
        scratch_types=[
            pltpu.VMEM((rows_per_w,), jnp.int32),
            pltpu.VMEM((R,), jnp.int32),
            pltpu.VMEM((R, H), jnp.float32),
            pltpu.VMEM((R, H), jnp.float32),
            pltpu.SemaphoreType.DMA,
        ],
    )
    def emb(ids_hbm, tok_hbm, pos_hbm, out_hbm, idx_v, pidx_v, tok_buf, pos_buf,
            sem):
        wid = lax.axis_index("s") * NC + lax.axis_index("c")
        base = wid * rows_per_w
        s0 = lax.rem(base, S)
        pltpu.sync_copy(ids_hbm.at[pl.ds(base, rows_per_w)], idx_v)

        def chunk_body(c, _):
            r0 = c * R
            pltpu.async_copy(
                tok_hbm.at[idx_v.at[pl.ds(r0, R)]], tok_buf, sem
            ).wait()
            pidx_v[...] = lax.iota(jnp.int32, R) + (s0 + POS_OFFSET + r0)
            pltpu.async_copy(pos_hbm.at[pidx_v], pos_buf, sem).wait()

            def add_row(r, _):
                def add_vec(j, _):
                    col = j * LANES
                    tok_buf[r, pl.ds(col, LANES)] = (
                        tok_buf[r, pl.ds(col, LANES)]
                        + pos_buf[r, pl.ds(col, LANES)]
                    )
                    return _
                return lax.fori_loop(0, vecs_per_row, add_vec, _)

            lax.fori_loop(0, R, add_row, None)
            pltpu.sync_copy(tok_buf, out_hbm.at[pl.ds(base + r0, R)])
            return _

        lax.fori_loop(0, n_chunks, chunk_body, None)

    return emb


def kernel(token_ids, token_table, pos_table):
    B, S = token_ids.shape
    V, H = token_table.shape
    info = plsc.get_sparse_core_info()
    emb = _make_kernel(B, S, V, H, info.num_cores, info.num_subcores)
    ids_flat = token_ids.reshape(B * S)
    out = emb(ids_flat, token_table, pos_table)
    return out.reshape(B, S, H)


# speedup vs baseline: 2.2995x; 2.2995x over previous
import functools

import jax
import jax.numpy as jnp
from jax import lax
from jax.experimental import pallas as pl
from jax.experimental.pallas import tpu as pltpu
from jax.experimental.pallas import tpu_sc as plsc

POS_OFFSET = 2
LANES = 16


@functools.lru_cache(maxsize=None)
def _make_kernel(B, S, V, H, NC, NS):
    NW = NC * NS
    PB = 2
    PAIRS = B // PB
    WPP = NW // PAIRS
    SW = S // WPP
    C = 8
    NCH = SW // C
    ROWS = PB * C
    rows_per_w = PB * SW
    total_rows = B * S
    vecs_per_row = H // LANES

    mesh = plsc.VectorSubcoreMesh(core_axis_name="c", subcore_axis_name="s")

    @functools.partial(
        pl.kernel,
        mesh=mesh,
        out_type=jax.ShapeDtypeStruct((total_rows, H), jnp.float32),
        scratch_types=[
            pltpu.VMEM((rows_per_w,), jnp.int32),
            pltpu.VMEM((LANES,), jnp.int32),
            pltpu.VMEM((LANES,), jnp.int32),
            pltpu.VMEM((ROWS, H), jnp.float32),
            pltpu.VMEM((ROWS, H), jnp.float32),
            pltpu.VMEM((C, H), jnp.float32),
            pltpu.VMEM((C, H), jnp.float32),
            pltpu.SemaphoreType.DMA,
            pltpu.SemaphoreType.DMA,
            pltpu.SemaphoreType.DMA,
            pltpu.SemaphoreType.DMA,
        ],
    )
    def emb(ids_hbm, tok_hbm, pos_hbm, out_hbm, idx_v, pidx0, pidx1,
            tok0, tok1, pos0, pos1, ts0, ts1, ps0, ps1):
        wid = lax.axis_index("s") * NC + lax.axis_index("c")
        pair = wid // WPP
        s0 = (wid % WPP) * SW
        pltpu.sync_copy(ids_hbm.at[pl.ds(wid * rows_per_w, rows_per_w)], idx_v)

        pidx = (pidx0, pidx1)
        toks = (tok0, tok1)
        poss = (pos0, pos1)
        tsem = (ts0, ts1)
        psem = (ps0, ps1)

        def issue(g, b):
            pltpu.async_copy(
                tok_hbm.at[idx_v.at[pl.ds(g * ROWS, ROWS)]], toks[b], tsem[b]
            )
            pidx[b][...] = lax.iota(jnp.int32, LANES) + (s0 + POS_OFFSET + g * C)
            pltpu.async_copy(
                pos_hbm.at[pidx[b].at[pl.ds(0, C)]], poss[b], psem[b]
            )

        issue(0, 0)
        issue(1, 1)

        def body(i, _):
            for b in range(2):
                g = 2 * i + b
                tok_b, pos_b = toks[b], poss[b]
                pltpu.make_async_copy(
                    tok_hbm.at[idx_v.at[pl.ds(g * ROWS, ROWS)]], tok_b, tsem[b]
                ).wait()
                pltpu.make_async_copy(
                    pos_hbm.at[pidx[b].at[pl.ds(0, C)]], pos_b, psem[b]
                ).wait()

                def add_row(r, _, tok_b=tok_b, pos_b=pos_b):
                    def add_vec(j, _):
                        col = j * LANES
                        pvec = pos_b[r, pl.ds(col, LANES)]
                        for b2 in range(PB):
                            plsc.addupdate(
                                tok_b.at[b2 * C + r, pl.ds(col, LANES)], pvec
                            )
                        return _
                    return lax.fori_loop(0, vecs_per_row, add_vec, _)

                lax.fori_loop(0, C, add_row, None)

                for b2 in range(PB):
                    row = (pair * PB + b2) * S + s0 + g * C
                    pltpu.sync_copy(
                        tok_b.at[pl.ds(b2 * C, C)], out_hbm.at[pl.ds(row, C)]
                    )

                @pl.when(g + 2 < NCH)
                def _issue(g=g, b=b):
                    issue(g + 2, b)

            return _

        lax.fori_loop(0, NCH // 2, body, None)

    return emb


def kernel(token_ids, token_table, pos_table):
    B, S = token_ids.shape
    V, H = token_table.shape
    info = plsc.get_sparse_core_info()
    NC, NS = info.num_cores, info.num_subcores
    NW = NC * NS
    PB = 2
    WPP = NW // (B // PB)
    SW = S // WPP
    C = 8
    ids_perm = (
        token_ids.reshape(B // PB, PB, WPP, SW // C, C)
        .transpose(0, 2, 3, 1, 4)
        .reshape(B * S)
    )
    emb = _make_kernel(B, S, V, H, NC, NS)
    out = emb(ids_perm, token_table, pos_table)
    return out.reshape(B, S, H)
